# baseline (speedup 1.0000x reference)
import functools

import jax
import jax.numpy as jnp
from jax import lax
from jax.experimental import pallas as pl
from jax.experimental.pallas import tpu as pltpu
from jax.experimental.pallas import tpu_sc as plsc

_NC = 2
_NS = 16
_NW = _NC * _NS
_GRP = 10


def _make_gather(batch, timesteps, embed_dim):
  total = batch * timesteps
  n_units = total // 128
  u_per_w = n_units // _NW
  n_groups = u_per_w // _GRP
  assert n_groups % 2 == 0
  n_eg = embed_dim // 8
  n_bg = batch // 128
  mesh = plsc.VectorSubcoreMesh(core_axis_name="c", subcore_axis_name="s")

  @functools.partial(
      pl.kernel,
      mesh=mesh,
      out_type=jax.ShapeDtypeStruct((timesteps, n_eg, n_bg, 8, 128),
                                    jnp.float32),
      scratch_types=[
          pltpu.VMEM((n_groups, _GRP * 128), jnp.int32),
          pltpu.VMEM((_GRP * 128, embed_dim), jnp.float32),
          pltpu.VMEM((_GRP * 128, embed_dim), jnp.float32),
          pltpu.VMEM((embed_dim, 128), jnp.float32),
          pltpu.VMEM((embed_dim, 128), jnp.float32),
          pltpu.SemaphoreType.DMA,
          pltpu.SemaphoreType.DMA,
          pltpu.SemaphoreType.DMA,
          pltpu.SemaphoreType.DMA,
      ],
      compiler_params=pltpu.CompilerParams(
          use_tc_tiling_on_sc=False, needs_layout_passes=False),
  )
  def gather_kernel(idx_hbm, table_hbm, out_hbm, idx_v, rows_0, rows_1,
                    tbuf_0, tbuf_1, sem_g0, sem_g1, sem_w0, sem_w1):
    wid = lax.axis_index("s") * _NC + lax.axis_index("c")
    u_base = wid * u_per_w
    rows = (rows_0, rows_1)
    tbuf = (tbuf_0, tbuf_1)
    sem_g = (sem_g0, sem_g1)
    sem_w = (sem_w0, sem_w1)

    pltpu.sync_copy(idx_hbm.at[wid], idx_v)

    def gather_start(j, b):
      pltpu.async_copy(table_hbm.at[idx_v.at[j]], rows[b], sem_g[b])

    def gather_wait(j, b):
      pltpu.make_async_copy(table_hbm.at[idx_v.at[j]], rows[b],
                            sem_g[b]).wait()

    def tile_pos(u):
      return u // n_bg, u % n_bg

    def write_start(u, p):
      t, bg = tile_pos(u)
      for eg in range(n_eg):
        pltpu.async_copy(tbuf[p].at[pl.ds(eg * 8, 8)], out_hbm.at[t, eg, bg],
                         sem_w[p])

    def write_wait(u, p):
      t, bg = tile_pos(u)
      for eg in range(n_eg):
        pltpu.make_async_copy(tbuf[p].at[pl.ds(eg * 8, 8)],
                              out_hbm.at[t, eg, bg], sem_w[p]).wait()

    def unit(j, kk, p, b):
      u = u_base + j * _GRP + kk
      lane = lax.iota(jnp.int32, 16)
      cols = [cb * 16 + ((lane + d) & 15)
              for cb in range(embed_dim // 16) for d in range(16)]

      @pl.when(j * _GRP + kk >= 2)
      def _():
        write_wait(u - 2, p)

      r0 = kk * 128

      def iblock(ib, carry):
        row_ids = r0 + ib * 16 + lane
        dst_col = ib * 16 + lane
        for col in cols:
          vals = plsc.load_gather(rows[b], [row_ids, col])
          plsc.store_scatter(tbuf[p], [col, dst_col], vals)
        return carry

      lax.fori_loop(0, 8, iblock, 0)
      write_start(u, p)

    def step(j, b):
      @pl.when(j + 1 < n_groups)
      def _():
        gather_start(j + 1, 1 - b)

      gather_wait(j, b)

      def pair(kp, carry):
        unit(j, 2 * kp, 0, b)
        unit(j, 2 * kp + 1, 1, b)
        return carry

      lax.fori_loop(0, _GRP // 2, pair, 0)

    gather_start(0, 0)

    def body(jj, carry):
      step(2 * jj, 0)
      step(2 * jj + 1, 1)
      return carry

    lax.fori_loop(0, n_groups // 2, body, 0)
    write_wait(u_base + u_per_w - 2, 0)
    write_wait(u_base + u_per_w - 1, 1)

  return gather_kernel


def _make_depad(vocab, embed_dim):
  chunk = 248
  v_main = vocab - vocab % (chunk * _NW)
  v_per_w = v_main // _NW
  n_chunks = v_per_w // chunk
  tail = vocab - v_main
  assert n_chunks % 2 == 0 and tail % 8 == 0 and tail <= chunk
  mesh = plsc.VectorSubcoreMesh(core_axis_name="c", subcore_axis_name="s")

  @functools.partial(
      pl.kernel,
      mesh=mesh,
      out_type=jax.ShapeDtypeStruct((vocab * embed_dim,), jnp.float32),
      scratch_types=[
          pltpu.VMEM((chunk, embed_dim), jnp.float32),
          pltpu.VMEM((chunk, embed_dim), jnp.float32),
          pltpu.VMEM((chunk * embed_dim,), jnp.float32),
          pltpu.VMEM((chunk * embed_dim,), jnp.float32),
          pltpu.SemaphoreType.DMA,
          pltpu.SemaphoreType.DMA,
          pltpu.SemaphoreType.DMA,
          pltpu.SemaphoreType.DMA,
      ],
      compiler_params=pltpu.CompilerParams(needs_layout_passes=False),
  )
  def depad_kernel(table_hbm, out_hbm, in_0, in_1, fl_0, fl_1,
                   sem_i0, sem_i1, sem_o0, sem_o1):
    wid = lax.axis_index("s") * _NC + lax.axis_index("c")
    r_base = wid * v_per_w
    inb = (in_0, in_1)
    flb = (fl_0, fl_1)
    sem_i = (sem_i0, sem_i1)
    sem_o = (sem_o0, sem_o1)

    def in_start(r0, n, b):
      pltpu.async_copy(table_hbm.at[pl.ds(r0, n)], inb[b].at[pl.ds(0, n)],
                       sem_i[b])

    def in_wait(r0, n, b):
      pltpu.make_async_copy(table_hbm.at[pl.ds(r0, n)],
                            inb[b].at[pl.ds(0, n)], sem_i[b]).wait()

    def out_start(r0, n, b):
      pltpu.async_copy(flb[b].at[pl.ds(0, n * embed_dim)],
                       out_hbm.at[pl.ds(r0 * embed_dim, n * embed_dim)],
                       sem_o[b])

    def out_wait(r0, n, b):
      pltpu.make_async_copy(flb[b].at[pl.ds(0, n * embed_dim)],
                            out_hbm.at[pl.ds(r0 * embed_dim, n * embed_dim)],
                            sem_o[b]).wait()

    def repack(n, b):
      def rrow(i, carry):
        for c16 in range(embed_dim // 16):
          flb[b][pl.ds(i * embed_dim + c16 * 16, 16)] = (
              inb[b][i, pl.ds(c16 * 16, 16)])
        return carry

      lax.fori_loop(0, n, rrow, 0)

    in_start(r_base, chunk, 0)

    def step(g, b):
      @pl.when(g + 1 < n_chunks)
      def _():
        in_start(r_base + (g + 1) * chunk, chunk, 1 - b)

      in_wait(r_base + g * chunk, chunk, b)

      @pl.when(g >= 2)
      def _():
        out_wait(r_base + (g - 2) * chunk, chunk, b)

      repack(chunk, b)
      out_start(r_base + g * chunk, chunk, b)

    def body(jj, carry):
      step(2 * jj, 0)
      step(2 * jj + 1, 1)
      return carry

    lax.fori_loop(0, n_chunks // 2, body, 0)
    out_wait(r_base + (n_chunks - 2) * chunk, chunk, 0)
    out_wait(r_base + (n_chunks - 1) * chunk, chunk, 1)

    if tail:
      @pl.when(wid == _NW - 1)
      def _():
        in_start(v_main, tail, 0)
        in_wait(v_main, tail, 0)
        repack(tail, 0)
        out_start(v_main, tail, 0)
        out_wait(v_main, tail, 0)

  return depad_kernel


def kernel(x, table):
  batch, timesteps = x.shape
  vocab, embed_dim = table.shape
  total = batch * timesteps
  assert batch % 128 == 0 and embed_dim % 8 == 0
  assert total % (_NW * _GRP * 128) == 0
  n_groups = total // (_NW * _GRP * 128)
  idx = x.T.reshape(_NW, n_groups, _GRP * 128).astype(jnp.int32)
  table_lin = _make_depad(vocab, embed_dim)(table).reshape(vocab, embed_dim)
  out5 = _make_gather(batch, timesteps, embed_dim)(idx, table_lin)
  return out5.transpose(2, 4, 0, 1, 3).reshape(batch, timesteps, embed_dim)


# speedup vs baseline: 1.5769x; 1.5769x over previous
import functools

import jax
import jax.numpy as jnp
from jax import lax
from jax.experimental import pallas as pl
from jax.experimental.pallas import tpu as pltpu
from jax.experimental.pallas import tpu_sc as plsc

_NC = 2
_NS = 16
_NW = _NC * _NS
_GRP = 10


def _make_gather(batch, timesteps, embed_dim):
  total = batch * timesteps
  n_units = total // 128
  u_per_w = n_units // _NW
  n_groups = u_per_w // _GRP
  assert n_groups % 2 == 0
  n_eg = embed_dim // 8
  n_bg = batch // 128
  mesh = plsc.VectorSubcoreMesh(core_axis_name="c", subcore_axis_name="s")

  @functools.partial(
      pl.kernel,
      mesh=mesh,
      out_type=jax.ShapeDtypeStruct((timesteps, n_eg, n_bg, 8, 128),
                                    jnp.float32),
      scratch_types=[
          pltpu.VMEM((n_groups, _GRP * 128), jnp.int32),
          pltpu.VMEM((_GRP * 128, embed_dim), jnp.float32),
          pltpu.VMEM((_GRP * 128, embed_dim), jnp.float32),
          pltpu.VMEM((embed_dim, 128), jnp.float32),
          pltpu.VMEM((embed_dim, 128), jnp.float32),
          pltpu.SemaphoreType.DMA,
          pltpu.SemaphoreType.DMA,
          pltpu.SemaphoreType.DMA,
          pltpu.SemaphoreType.DMA,
      ],
      compiler_params=pltpu.CompilerParams(
          use_tc_tiling_on_sc=False, needs_layout_passes=False),
  )
  def gather_kernel(idx_hbm, table_hbm, out_hbm, idx_v, rows_0, rows_1,
                    tbuf_0, tbuf_1, sem_g0, sem_g1, sem_w0, sem_w1):
    wid = lax.axis_index("s") * _NC + lax.axis_index("c")
    u_base = wid * u_per_w
    rows = (rows_0, rows_1)
    tbuf = (tbuf_0, tbuf_1)
    sem_g = (sem_g0, sem_g1)
    sem_w = (sem_w0, sem_w1)

    pltpu.sync_copy(idx_hbm.at[wid], idx_v)

    def gather_start(j, b):
      pltpu.async_copy(table_hbm.at[idx_v.at[j]], rows[b], sem_g[b])

    def gather_wait(j, b):
      pltpu.make_async_copy(table_hbm.at[idx_v.at[j]], rows[b],
                            sem_g[b]).wait()

    def tile_pos(u):
      return u // n_bg, u % n_bg

    def write_start(u, p):
      t, bg = tile_pos(u)
      for eg in range(n_eg):
        pltpu.async_copy(tbuf[p].at[pl.ds(eg * 8, 8)], out_hbm.at[t, eg, bg],
                         sem_w[p])

    def write_wait(u, p):
      t, bg = tile_pos(u)
      for eg in range(n_eg):
        pltpu.make_async_copy(tbuf[p].at[pl.ds(eg * 8, 8)],
                              out_hbm.at[t, eg, bg], sem_w[p]).wait()

    def unit(j, kk, p, b):
      u = u_base + j * _GRP + kk
      lane = lax.iota(jnp.int32, 16)
      cols = [cb * 16 + ((lane + d) & 15)
              for cb in range(embed_dim // 16) for d in range(16)]

      @pl.when(j * _GRP + kk >= 2)
      def _():
        write_wait(u - 2, p)

      r0 = kk * 128

      def iblock(ib, carry):
        row_ids = r0 + ib * 16 + lane
        dst_col = ib * 16 + lane
        for col in cols:
          vals = plsc.load_gather(rows[b], [row_ids, col])
          plsc.store_scatter(tbuf[p], [col, dst_col], vals)
        return carry

      lax.fori_loop(0, 8, iblock, 0)
      write_start(u, p)

    def step(j, b):
      @pl.when(j + 1 < n_groups)
      def _():
        gather_start(j + 1, 1 - b)

      gather_wait(j, b)

      def pair(kp, carry):
        unit(j, 2 * kp, 0, b)
        unit(j, 2 * kp + 1, 1, b)
        return carry

      lax.fori_loop(0, _GRP // 2, pair, 0)

    gather_start(0, 0)

    def body(jj, carry):
      step(2 * jj, 0)
      step(2 * jj + 1, 1)
      return carry

    lax.fori_loop(0, n_groups // 2, body, 0)
    write_wait(u_base + u_per_w - 2, 0)
    write_wait(u_base + u_per_w - 1, 1)

  return gather_kernel


def _make_depad(vocab, embed_dim):
  chunk = 256
  v_per_w = (vocab // (chunk * _NW)) * chunk
  n_chunks = v_per_w // chunk
  v_main = v_per_w * _NW
  n_extra = (vocab - v_main) // chunk
  tail = vocab - v_main - n_extra * chunk
  assert n_chunks % 2 == 0 and embed_dim % 16 == 0 and tail % 8 == 0
  mesh = plsc.VectorSubcoreMesh(core_axis_name="c", subcore_axis_name="s")

  @functools.partial(
      pl.kernel,
      mesh=mesh,
      out_type=jax.ShapeDtypeStruct((vocab * embed_dim,), jnp.float32),
      scratch_types=[
          pltpu.VMEM((max(tail, 8), embed_dim), jnp.float32),
          pltpu.VMEM((embed_dim, chunk), jnp.float32),
          pltpu.VMEM((embed_dim, chunk), jnp.float32),
          pltpu.VMEM((chunk * embed_dim,), jnp.float32),
          pltpu.VMEM((chunk * embed_dim,), jnp.float32),
          pltpu.SemaphoreType.DMA,
          pltpu.SemaphoreType.DMA,
          pltpu.SemaphoreType.DMA,
          pltpu.SemaphoreType.DMA,
      ],
      compiler_params=pltpu.CompilerParams(needs_layout_passes=False),
  )
  def depad_kernel(tableT_hbm, tail_hbm, out_hbm, tail_v, in_0, in_1,
                   fl_0, fl_1, sem_i0, sem_i1, sem_o0, sem_o1):
    wid = lax.axis_index("s") * _NC + lax.axis_index("c")
    v_base = wid * v_per_w
    inb = (in_0, in_1)
    flb = (fl_0, fl_1)
    sem_i = (sem_i0, sem_i1)
    sem_o = (sem_o0, sem_o1)
    lane = lax.iota(jnp.int32, 16)
    wrapped = [(lane + d) & 15 for d in range(16)]
    wrapped32 = [w * embed_dim for w in wrapped]

    def in_start(v0, n, b):
      pltpu.async_copy(tableT_hbm.at[:, pl.ds(v0, n)],
                       inb[b].at[:, pl.ds(0, n)], sem_i[b])

    def in_wait(v0, n, b):
      pltpu.make_async_copy(tableT_hbm.at[:, pl.ds(v0, n)],
                            inb[b].at[:, pl.ds(0, n)], sem_i[b]).wait()

    def out_start(v0, n, b):
      pltpu.async_copy(flb[b].at[pl.ds(0, n * embed_dim)],
                       out_hbm.at[pl.ds(v0 * embed_dim, n * embed_dim)],
                       sem_o[b])

    def out_wait(v0, n, b):
      pltpu.make_async_copy(flb[b].at[pl.ds(0, n * embed_dim)],
                            out_hbm.at[pl.ds(v0 * embed_dim, n * embed_dim)],
                            sem_o[b]).wait()

    def transpose(n, b):
      def vblock(vb, carry):
        for eh in range(embed_dim // 16):
          row_e = eh * 16 + lane
          base = vb * (16 * embed_dim) + eh * 16 + lane
          for d in range(16):
            col_v = vb * 16 + wrapped[d]
            vals = plsc.load_gather(inb[b], [row_e, col_v])
            plsc.store_scatter(flb[b], [base + wrapped32[d]], vals)
        return carry

      lax.fori_loop(0, n // 16, vblock, 0)

    in_start(v_base, chunk, 0)

    def step(g, b):
      @pl.when(g + 1 < n_chunks)
      def _():
        in_start(v_base + (g + 1) * chunk, chunk, 1 - b)

      in_wait(v_base + g * chunk, chunk, b)

      @pl.when(g >= 2)
      def _():
        out_wait(v_base + (g - 2) * chunk, chunk, b)

      transpose(chunk, b)
      out_start(v_base + g * chunk, chunk, b)

    def body(jj, carry):
      step(2 * jj, 0)
      step(2 * jj + 1, 1)
      return carry

    lax.fori_loop(0, n_chunks // 2, body, 0)
    out_wait(v_base + (n_chunks - 2) * chunk, chunk, 0)
    out_wait(v_base + (n_chunks - 1) * chunk, chunk, 1)

    @pl.when(wid == _NW - 1)
    def _():
      for k in range(n_extra):
        v0 = v_main + k * chunk
        in_start(v0, chunk, 0)
        in_wait(v0, chunk, 0)
        transpose(chunk, 0)
        out_start(v0, chunk, 0)
        out_wait(v0, chunk, 0)
      if tail:
        v0 = vocab - tail
        pltpu.sync_copy(tail_hbm, tail_v)

        def trow(i, carry):
          for c16 in range(embed_dim // 16):
            flb[0][pl.ds(i * embed_dim + c16 * 16, 16)] = (
                tail_v[i, pl.ds(c16 * 16, 16)])
          return carry

        lax.fori_loop(0, tail, trow, 0)
        out_start(v0, tail, 0)
        out_wait(v0, tail, 0)

  return depad_kernel


def kernel(x, table):
  batch, timesteps = x.shape
  vocab, embed_dim = table.shape
  total = batch * timesteps
  assert batch % 128 == 0 and embed_dim % 8 == 0
  assert total % (_NW * _GRP * 128) == 0
  n_groups = total // (_NW * _GRP * 128)
  idx = x.T.reshape(_NW, n_groups, _GRP * 128).astype(jnp.int32)
  v_tail = vocab - (vocab // (256 * _NW)) * 256 * _NW
  v_tail -= (v_tail // 256) * 256
  table_lin = _make_depad(vocab, embed_dim)(
      table.T, table[vocab - v_tail:]).reshape(vocab, embed_dim)
  out5 = _make_gather(batch, timesteps, embed_dim)(idx, table_lin)
  return out5.transpose(2, 4, 0, 1, 3).reshape(batch, timesteps, embed_dim)
